# baseline (speedup 1.0000x reference)
import functools

import jax
import jax.numpy as jnp
from jax import lax
from jax.experimental import pallas as pl
from jax.experimental.pallas import tpu as pltpu
from jax.experimental.pallas import tpu_sc as plsc

BS = 32
C = 128
NPT = 129
NPAD = 144
N_KEEP = 5
BIG = 1e30
T = 0.07
KCHUNK = 18816
KSTEPS = 150528 // KCHUNK


def _encode_graph_body(xq_ref, xk_ref, w_ref, b_ref, queue_ref,
                       adj_ref, qd_ref, lpos_ref, acc_ref, adj_v, sem):
    k = pl.program_id(0)

    @pl.when(k == 0)
    def _init():
        acc_ref[...] = jnp.zeros_like(acc_ref)

    w = w_ref[...]
    acc_ref[:BS, :] += jnp.dot(xq_ref[...], w,
                               preferred_element_type=jnp.float32)
    acc_ref[BS:, :] += jnp.dot(xk_ref[...], w,
                               preferred_element_type=jnp.float32)

    @pl.when(k == KSTEPS - 1)
    def _epilogue():
        y = acc_ref[...] + b_ref[...]
        n = jnp.sqrt(jnp.sum(y * y, axis=1, keepdims=True))
        qk = y / jnp.maximum(n, 1e-12)
        _graph_math(qk, queue_ref[...], adj_v, qd_ref, lpos_ref)
        cp = pltpu.make_async_copy(adj_v, adj_ref, sem)
        cp.start()
        cp.wait()


def _graph_math(qk, queue, adj_ref, qd_ref, lpos_ref):
    q = qk[:BS, :]
    kk = qk[BS:, :]
    kq = queue.T

    dqq = jnp.sum((kq[:, None, :] - kq[None, :, :]) ** 2, axis=-1)
    dq2 = jnp.sum((q[:, None, :] - kq[None, :, :]) ** 2, axis=-1)
    eqq = jnp.sqrt(dqq)
    eq = jnp.sqrt(dq2)

    neg = jnp.float32(-1e30)
    dead = jnp.float32(-3e38)
    bigf = jnp.float32(BIG)

    f = jnp.concatenate(
        [jnp.full((C, 1), neg, jnp.float32), eqq,
         jnp.full((C, NPAD - NPT), neg, jnp.float32)], axis=1)
    idx2 = lax.broadcasted_iota(jnp.int32, (C, NPAD), 1)
    workf = f
    masks = []
    for _ in range(N_KEEP):
        cm = jnp.max(workf, axis=1, keepdims=True)
        sel = jnp.max(jnp.where(workf == cm, idx2, jnp.int32(-1)), axis=1,
                      keepdims=True)
        hit = idx2 == sel
        masks.append(hit)
        workf = jnp.where(hit, dead, workf)
    mask_a = masks[0] | masks[1] | masks[2] | masks[3]
    mask_5 = masks[4]
    f5v = jnp.max(jnp.where(mask_5, f, dead), axis=1)
    in_flag = eq > f5v[None, :]

    keepf = mask_a[None] | (mask_5[None] & ~in_flag[:, :, None])
    vals = jnp.broadcast_to(f[None], (BS, C, NPAD))
    adj_b = jnp.where(keepf & (vals > 0), vals, bigf)
    idx_n = lax.broadcasted_iota(jnp.int32, (BS, C, NPAD), 2)
    q_col = jnp.where((in_flag & (eq > 0))[:, :, None], eq[:, :, None], bigf)
    adj_b = jnp.where(idx_n == 0, q_col, adj_b)

    row0 = jnp.concatenate(
        [jnp.zeros((BS, 1), jnp.float32), eq,
         jnp.full((BS, NPAD - NPT), neg, jnp.float32)], axis=1)
    idx_r = lax.broadcasted_iota(jnp.int32, (BS, NPAD), 1)
    w0 = row0
    kept0 = jnp.zeros((BS, NPAD), dtype=jnp.bool_)
    for _ in range(N_KEEP):
        cm = jnp.max(w0, axis=1, keepdims=True)
        sel = jnp.max(jnp.where(w0 == cm, idx_r, jnp.int32(-1)), axis=1,
                      keepdims=True)
        hit = idx_r == sel
        kept0 = kept0 | hit
        w0 = jnp.where(hit, dead, w0)
    adj_0 = jnp.where(kept0 & (row0 > 0), row0, bigf)

    adj_ref[...] = jnp.concatenate([adj_0[:, None, :], adj_b], axis=1)
    qd_ref[...] = jnp.dot(q, queue, preferred_element_type=jnp.float32)
    lpos_ref[...] = jnp.sum(q * kk, axis=1, keepdims=True)


def _encode_and_graph(xq, xk, w, b2, queue):
    return pl.pallas_call(
        _encode_graph_body,
        grid=(KSTEPS,),
        in_specs=[
            pl.BlockSpec((BS, KCHUNK), lambda k: (k * 0, k)),
            pl.BlockSpec((BS, KCHUNK), lambda k: (k * 0, k)),
            pl.BlockSpec((KCHUNK, C), lambda k: (k, k * 0)),
            pl.BlockSpec((1, C), lambda k: (k * 0, k * 0)),
            pl.BlockSpec((C, C), lambda k: (k * 0, k * 0)),
        ],
        out_specs=(
            pl.BlockSpec(memory_space=pl.ANY),
            pl.BlockSpec((BS, C), lambda k: (k * 0, k * 0)),
            pl.BlockSpec((BS, 1), lambda k: (k * 0, k * 0)),
        ),
        out_shape=(
            jax.ShapeDtypeStruct((BS, NPT, NPAD), jnp.float32),
            jax.ShapeDtypeStruct((BS, C), jnp.float32),
            jax.ShapeDtypeStruct((BS, 1), jnp.float32),
        ),
        scratch_shapes=[
            pltpu.VMEM((2 * BS, C), jnp.float32),
            pltpu.VMEM((BS, NPT, NPAD), jnp.float32),
            pltpu.SemaphoreType.DMA,
        ],
        compiler_params=pltpu.CompilerParams(
            vmem_limit_bytes=100 * 1024 * 1024),
    )(xq, xk, w, b2, queue)


NCHUNK = NPAD // 16


def _dijkstra_body(adj_hbm, out_hbm, a_v, dist_v, vis_v, dmask_v):
    wid = lax.axis_index("s") * 2 + lax.axis_index("c")
    pltpu.sync_copy(adj_hbm.at[wid], a_v)

    lane = lax.iota(jnp.int32, 16)
    for j in range(NCHUNK):
        lj = lane + 16 * j
        d0 = jnp.where(lj == 0, jnp.float32(0.0), jnp.float32(BIG))
        v0 = jnp.where(lj < NPT, jnp.float32(0.0), jnp.float32(BIG))
        dist_v[pl.ds(16 * j, 16)] = d0
        vis_v[pl.ds(16 * j, 16)] = v0
        dmask_v[pl.ds(16 * j, 16)] = d0 + v0

    def step(_, carry):
        mvec = dmask_v[pl.ds(0, 16)]
        bvec = jnp.zeros((16,), jnp.int32)
        for j in range(1, NCHUNK):
            cj = dmask_v[pl.ds(16 * j, 16)]
            lt = cj < mvec
            mvec = jnp.where(lt, cj, mvec)
            bvec = jnp.where(lt, jnp.int32(j), bvec)
        m = jnp.min(mvec)
        v = jnp.min(jnp.where(mvec == m, bvec * 16 + lane, jnp.int32(10_000)))
        for j in range(NCHUNK):
            row = a_v[v, pl.ds(16 * j, 16)]
            dj = jnp.minimum(dist_v[pl.ds(16 * j, 16)], m + row)
            lj = lane + 16 * j
            vj = jnp.where(lj == v, jnp.float32(BIG), vis_v[pl.ds(16 * j, 16)])
            dist_v[pl.ds(16 * j, 16)] = dj
            vis_v[pl.ds(16 * j, 16)] = vj
            dmask_v[pl.ds(16 * j, 16)] = dj + vj
        return carry

    lax.fori_loop(jnp.int32(0), jnp.int32(NPT), step, jnp.int32(0))
    pltpu.sync_copy(dist_v, out_hbm.at[wid])


def _dijkstra_sc(adj):
    mesh = plsc.VectorSubcoreMesh(core_axis_name="c", subcore_axis_name="s")
    f = pl.kernel(
        _dijkstra_body,
        out_type=jax.ShapeDtypeStruct((BS, NPAD), jnp.float32),
        mesh=mesh,
        scratch_types=[
            pltpu.VMEM((NPT, NPAD), jnp.float32),
            pltpu.VMEM((NPAD,), jnp.float32),
            pltpu.VMEM((NPAD,), jnp.float32),
            pltpu.VMEM((NPAD,), jnp.float32),
        ],
        compiler_params=pltpu.CompilerParams(needs_layout_passes=False),
    )
    return f(adj)


def _finish_body(dist_ref, qd_ref, lpos_ref, out_ref):
    d = dist_ref[...]
    col = lax.broadcasted_iota(jnp.int32, (BS, NPAD), 1)
    valid = (col >= 1) & (col < NPT)
    reach = d < jnp.float32(1e29)
    mx = jnp.max(jnp.where(valid & reach, d, jnp.float32(0.0)))
    newd = jnp.where(reach, d, mx + 1.0)
    wgt = 1.0 / (1.0 + newd)
    wk = wgt[:, 1:NPT]
    logits = jnp.concatenate([lpos_ref[...], qd_ref[...] * wk], axis=1) / T
    out_ref[...] = logits


def _finish(dist, qdots, lpos):
    return pl.pallas_call(
        _finish_body,
        out_shape=jax.ShapeDtypeStruct((BS, NPT), jnp.float32),
    )(dist, qdots, lpos)


def kernel(img_q, img_k, Wq, bq, queue):
    adj, qdots, lpos = _encode_and_graph(
        img_q.reshape(BS, -1), img_k.reshape(BS, -1),
        Wq, bq.reshape(1, C), queue)
    dist = _dijkstra_sc(adj)
    logits = _finish(dist, qdots, lpos)
    labels = jnp.zeros((BS,), dtype=jnp.int32)
    return logits, labels


# speedup vs baseline: 1.0419x; 1.0419x over previous
import functools

import jax
import jax.numpy as jnp
from jax import lax
from jax.experimental import pallas as pl
from jax.experimental.pallas import tpu as pltpu
from jax.experimental.pallas import tpu_sc as plsc

BS = 32
C = 128
NPT = 129
NPAD = 144
N_KEEP = 5
BIG = 1e30
T = 0.07
KCHUNK = 12544
KSTEPS = 150528 // KCHUNK
NE = 16
ND = 160


def _encode_graph_body(xq_ref, xk_ref, w_ref, b_ref, queue_ref,
                       ei_ref, ev_ref, qd_ref, lpos_ref,
                       acc_ref, ei_v, ev_v, sem1, sem2):
    k = pl.program_id(0)

    @pl.when(k == 0)
    def _init():
        acc_ref[...] = jnp.zeros_like(acc_ref)

    w = w_ref[...]
    acc_ref[:BS, :] += jnp.dot(xq_ref[...], w,
                               preferred_element_type=jnp.float32)
    acc_ref[BS:, :] += jnp.dot(xk_ref[...], w,
                               preferred_element_type=jnp.float32)

    @pl.when(k == KSTEPS - 1)
    def _epilogue():
        y = acc_ref[...] + b_ref[...]
        n = jnp.sqrt(jnp.sum(y * y, axis=1, keepdims=True))
        qk = y / jnp.maximum(n, 1e-12)
        _graph_math(qk, queue_ref[...], ei_v, ev_v, qd_ref, lpos_ref)
        cp1 = pltpu.make_async_copy(ei_v, ei_ref, sem1)
        cp2 = pltpu.make_async_copy(ev_v, ev_ref, sem2)
        cp1.start()
        cp2.start()
        cp1.wait()
        cp2.wait()


def _graph_math(qk, queue, ei_ref, ev_ref, qd_ref, lpos_ref):
    q = qk[:BS, :]
    kk = qk[BS:, :]
    kq = queue.T

    dqq = jnp.sum((kq[:, None, :] - kq[None, :, :]) ** 2, axis=-1)
    dq2 = jnp.sum((q[:, None, :] - kq[None, :, :]) ** 2, axis=-1)
    eqq = jnp.sqrt(dqq)
    eq = jnp.sqrt(dq2)

    neg = jnp.float32(-1e30)
    dead = jnp.float32(-3e38)
    bigf = jnp.float32(BIG)

    f = jnp.concatenate(
        [jnp.full((C, 1), neg, jnp.float32), eqq,
         jnp.full((C, NPAD - NPT), neg, jnp.float32)], axis=1)
    idx2 = lax.broadcasted_iota(jnp.int32, (C, NPAD), 1)
    workf = f
    sels, vals = [], []
    for _ in range(N_KEEP):
        cm = jnp.max(workf, axis=1, keepdims=True)
        sel = jnp.max(jnp.where(workf == cm, idx2, jnp.int32(-1)), axis=1,
                      keepdims=True)
        sels.append(sel)
        vals.append(cm)
        workf = jnp.where(idx2 == sel, dead, workf)
    fixed_idx = jnp.concatenate(sels, axis=1)
    fixed_val = jnp.concatenate(vals, axis=1)
    f5v = fixed_val[:, 4]
    in_flag = eq > f5v[None, :]

    fi4 = jnp.broadcast_to(fixed_idx[None, :, :4], (BS, C, 4))
    fv4 = jnp.broadcast_to(fixed_val[None, :, :4], (BS, C, 4))
    fv4 = jnp.where(fv4 > 0, fv4, bigf)
    i5 = jnp.broadcast_to(fixed_idx[None, :, 4:5], (BS, C, 1))
    v5 = jnp.broadcast_to(fixed_val[None, :, 4:5], (BS, C, 1))
    v5 = jnp.where((~in_flag)[:, :, None] & (v5 > 0), v5, bigf)
    iq = jnp.zeros((BS, C, 1), jnp.int32)
    vq = jnp.where((in_flag & (eq > 0))[:, :, None], eq[:, :, None], bigf)
    pad_i = lax.broadcasted_iota(jnp.int32, (BS, C, 10), 2) + jnp.int32(150)
    pad_v = jnp.full((BS, C, 10), bigf, jnp.float32)
    ei_body = jnp.concatenate([fi4, i5, iq, pad_i], axis=2)
    ev_body = jnp.concatenate([fv4, v5, vq, pad_v], axis=2)

    row0 = jnp.concatenate(
        [jnp.zeros((BS, 1), jnp.float32), eq,
         jnp.full((BS, NPAD - NPT), neg, jnp.float32)], axis=1)
    idx_r = lax.broadcasted_iota(jnp.int32, (BS, NPAD), 1)
    w0 = row0
    sels0, vals0 = [], []
    for _ in range(N_KEEP):
        cm = jnp.max(w0, axis=1, keepdims=True)
        sel = jnp.max(jnp.where(w0 == cm, idx_r, jnp.int32(-1)), axis=1,
                      keepdims=True)
        sels0.append(sel)
        vals0.append(cm)
        w0 = jnp.where(idx_r == sel, dead, w0)
    r0i = jnp.concatenate(sels0, axis=1)
    r0v = jnp.concatenate(vals0, axis=1)
    r0v = jnp.where(r0v > 0, r0v, bigf)
    pad0_i = lax.broadcasted_iota(jnp.int32, (BS, 11), 1) + jnp.int32(149)
    ei_0 = jnp.concatenate([r0i, pad0_i], axis=1)
    ev_0 = jnp.concatenate(
        [r0v, jnp.full((BS, 11), bigf, jnp.float32)], axis=1)

    ei_ref[...] = jnp.concatenate([ei_0[:, None, :], ei_body], axis=1)
    ev_ref[...] = jnp.concatenate([ev_0[:, None, :], ev_body], axis=1)
    qd_ref[...] = jnp.dot(q, queue, preferred_element_type=jnp.float32)
    lpos_ref[...] = jnp.sum(q * kk, axis=1, keepdims=True)


def _encode_and_graph(xq, xk, w, b2, queue):
    return pl.pallas_call(
        _encode_graph_body,
        grid=(KSTEPS,),
        in_specs=[
            pl.BlockSpec((BS, KCHUNK), lambda k: (k * 0, k)),
            pl.BlockSpec((BS, KCHUNK), lambda k: (k * 0, k)),
            pl.BlockSpec((KCHUNK, C), lambda k: (k, k * 0)),
            pl.BlockSpec((1, C), lambda k: (k * 0, k * 0)),
            pl.BlockSpec((C, C), lambda k: (k * 0, k * 0)),
        ],
        out_specs=(
            pl.BlockSpec(memory_space=pl.ANY),
            pl.BlockSpec(memory_space=pl.ANY),
            pl.BlockSpec((BS, C), lambda k: (k * 0, k * 0)),
            pl.BlockSpec((BS, 1), lambda k: (k * 0, k * 0)),
        ),
        out_shape=(
            jax.ShapeDtypeStruct((BS, NPT, NE), jnp.int32),
            jax.ShapeDtypeStruct((BS, NPT, NE), jnp.float32),
            jax.ShapeDtypeStruct((BS, C), jnp.float32),
            jax.ShapeDtypeStruct((BS, 1), jnp.float32),
        ),
        scratch_shapes=[
            pltpu.VMEM((2 * BS, C), jnp.float32),
            pltpu.VMEM((BS, NPT, NE), jnp.int32),
            pltpu.VMEM((BS, NPT, NE), jnp.float32),
            pltpu.SemaphoreType.DMA,
            pltpu.SemaphoreType.DMA,
        ],
        compiler_params=pltpu.CompilerParams(
            vmem_limit_bytes=100 * 1024 * 1024),
    )(xq, xk, w, b2, queue)


NCHUNK = ND // 16


def _dijkstra_body(ei_hbm, ev_hbm, out_hbm, ei_v, ev_v, dist_v, vis_v,
                   dmask_v):
    wid = lax.axis_index("s") * 2 + lax.axis_index("c")
    pltpu.sync_copy(ei_hbm.at[wid], ei_v)
    pltpu.sync_copy(ev_hbm.at[wid], ev_v)

    lane = lax.iota(jnp.int32, 16)
    bigv = jnp.full((16,), jnp.float32(BIG), jnp.float32)
    for j in range(NCHUNK):
        lj = lane + 16 * j
        d0 = jnp.where(lj == 0, jnp.float32(0.0), jnp.float32(BIG))
        v0 = jnp.where(lj < NPT, jnp.float32(0.0), jnp.float32(BIG))
        dist_v[pl.ds(16 * j, 16)] = d0
        vis_v[pl.ds(16 * j, 16)] = v0
        dmask_v[pl.ds(16 * j, 16)] = d0 + v0

    def step(_, carry):
        mvec = dmask_v[pl.ds(0, 16)]
        bvec = jnp.zeros((16,), jnp.int32)
        for j in range(1, NCHUNK):
            cj = dmask_v[pl.ds(16 * j, 16)]
            lt = cj < mvec
            mvec = jnp.where(lt, cj, mvec)
            bvec = jnp.where(lt, jnp.int32(j), bvec)
        m = jnp.min(mvec)
        v = jnp.min(jnp.where(mvec == m, bvec * 16 + lane, jnp.int32(10_000)))
        ei = ei_v[v, :]
        ev = ev_v[v, :]
        nd = m + ev
        cur = plsc.load_gather(dist_v, [ei])
        new = jnp.minimum(cur, nd)
        plsc.store_scatter(dist_v, [ei], new)
        visg = plsc.load_gather(vis_v, [ei])
        plsc.store_scatter(dmask_v, [ei], new + visg)
        vfull = jnp.full((16,), jnp.int32(0), jnp.int32) + v
        m0 = lane == 0
        plsc.store_scatter(vis_v, [vfull], bigv, mask=m0)
        plsc.store_scatter(dmask_v, [vfull], bigv, mask=m0)
        return carry

    lax.fori_loop(jnp.int32(0), jnp.int32(NPT), step, jnp.int32(0))
    pltpu.sync_copy(dist_v, out_hbm.at[wid])


def _dijkstra_sc(eidx, evals):
    mesh = plsc.VectorSubcoreMesh(core_axis_name="c", subcore_axis_name="s")
    f = pl.kernel(
        _dijkstra_body,
        out_type=jax.ShapeDtypeStruct((BS, ND), jnp.float32),
        mesh=mesh,
        scratch_types=[
            pltpu.VMEM((NPT, NE), jnp.int32),
            pltpu.VMEM((NPT, NE), jnp.float32),
            pltpu.VMEM((ND,), jnp.float32),
            pltpu.VMEM((ND,), jnp.float32),
            pltpu.VMEM((ND,), jnp.float32),
        ],
        compiler_params=pltpu.CompilerParams(needs_layout_passes=False),
    )
    return f(eidx, evals)


def _finish_body(dist_ref, qd_ref, lpos_ref, out_ref):
    d = dist_ref[...]
    col = lax.broadcasted_iota(jnp.int32, (BS, ND), 1)
    valid = (col >= 1) & (col < NPT)
    reach = d < jnp.float32(1e29)
    mx = jnp.max(jnp.where(valid & reach, d, jnp.float32(0.0)))
    newd = jnp.where(reach, d, mx + 1.0)
    wgt = 1.0 / (1.0 + newd)
    wk = wgt[:, 1:NPT]
    logits = jnp.concatenate([lpos_ref[...], qd_ref[...] * wk], axis=1) / T
    out_ref[...] = logits


def _finish(dist, qdots, lpos):
    return pl.pallas_call(
        _finish_body,
        out_shape=jax.ShapeDtypeStruct((BS, NPT), jnp.float32),
    )(dist, qdots, lpos)


def kernel(img_q, img_k, Wq, bq, queue):
    eidx, evals, qdots, lpos = _encode_and_graph(
        img_q.reshape(BS, -1), img_k.reshape(BS, -1),
        Wq, bq.reshape(1, C), queue)
    dist = _dijkstra_sc(eidx, evals)
    logits = _finish(dist, qdots, lpos)
    labels = jnp.zeros((BS,), dtype=jnp.int32)
    return logits, labels
